# Initial kernel scaffold; baseline (speedup 1.0000x reference)
import jax
import jax.numpy as jnp
from jax.experimental import pallas as pl


def kernel(coords, values):
    raise NotImplementedError("write your pallas kernel here")


# speedup vs baseline: 24.3814x; 24.3814x over previous
import functools

import jax
import jax.numpy as jnp
from jax import lax
from jax.experimental import pallas as pl
from jax.experimental.pallas import tpu as pltpu
from jax.experimental.pallas import tpu_sc as plsc

H = 16
W = 16
HW = H * W
B = 4
C = 16
BC = B * C
N = 262144
EPS = 1e-6
T0 = -1.0 - EPS
INV = H / (2.0 + 2 * EPS)

NC = 2
NS = 16
NW = NC * NS
PTS = N // NW
CHUNK = 512
NCHUNK = PTS // CHUNK
GROUPS = CHUNK // 16
SL = HW * BC // NS

_mesh = plsc.VectorSubcoreMesh(core_axis_name="c", subcore_axis_name="s")


@functools.partial(
    pl.kernel,
    out_type=(
        jax.ShapeDtypeStruct((NC, HW * BC), jnp.float32),
        jax.ShapeDtypeStruct((NC, HW), jnp.float32),
    ),
    mesh=_mesh,
    compiler_params=pltpu.CompilerParams(needs_layout_passes=False),
    scratch_types=[
        pltpu.VMEM((CHUNK * 2,), jnp.float32),
        pltpu.VMEM((B * CHUNK * C,), jnp.float32),
        pltpu.VMEM((HW * BC,), jnp.float32),
        pltpu.VMEM((16 * HW,), jnp.float32),
        pltpu.VMEM((NS, SL), jnp.float32),
        pltpu.VMEM((NS, HW), jnp.float32),
        pltpu.VMEM((SL,), jnp.float32),
        pltpu.VMEM((HW,), jnp.float32),
        pltpu.VMEM_SHARED((NS, HW * BC), jnp.float32),
        pltpu.VMEM_SHARED((NS, HW), jnp.float32),
    ],
)
def _pool_sc(coords_hbm, values_hbm, out_sums, out_cnts,
             cbuf, vbuf, acc, cntacc, red, cntstage, outbuf, cnt256,
             shared_acc, shared_cnt):
    cid = lax.axis_index("c")
    sid = lax.axis_index("s")
    wid = cid * NS + sid
    base = wid * PTS

    iota = lax.broadcasted_iota(jnp.int32, (16,), 0)
    zeros = jnp.zeros((16,), jnp.float32)
    ones = jnp.ones((16,), jnp.float32)

    def zero_acc(i, _):
        acc[pl.ds(i * 16, 16)] = zeros
        return 0
    lax.fori_loop(0, HW * BC // 16, zero_acc, 0)

    def zero_cnt(i, _):
        cntacc[pl.ds(i * 16, 16)] = zeros
        return 0
    lax.fori_loop(0, 16 * HW // 16, zero_cnt, 0)

    def chunk_body(k, _):
        off = base + k * CHUNK
        pltpu.sync_copy(coords_hbm.at[pl.ds(off * 2, CHUNK * 2)], cbuf)
        for b in range(B):
            pltpu.sync_copy(values_hbm.at[pl.ds((b * N + off) * C, CHUNK * C)],
                            vbuf.at[pl.ds(b * CHUNK * C, CHUNK * C)])

        def group_body(g, _):
            flat = g * 32 + iota * 2
            x = plsc.load_gather(cbuf, [flat])
            y = plsc.load_gather(cbuf, [flat + 1])
            bx = ((x - T0) * INV).astype(jnp.int32)
            by = ((y - T0) * INV).astype(jnp.int32)
            binv = bx + by * H
            plsc.addupdate_scatter(cntacc, [iota * HW + binv], ones)
            bofs = binv * BC
            for l in range(16):
                idx0 = iota + bofs[l]
                p = g * 16 + l
                for b in range(B):
                    v = vbuf[pl.ds((b * CHUNK + p) * C, C)]
                    plsc.addupdate_scatter(acc, [idx0 + b * C], v)
            return 0
        lax.fori_loop(0, GROUPS, group_body, 0)
        return 0

    lax.fori_loop(0, NCHUNK, chunk_body, 0)

    def cnt_red(j, _):
        s = zeros
        for l in range(16):
            s = s + cntacc[pl.ds(l * HW + j * 16, 16)]
        cnt256[pl.ds(j * 16, 16)] = s
        return 0
    lax.fori_loop(0, HW // 16, cnt_red, 0)

    pltpu.sync_copy(acc, shared_acc.at[sid])
    pltpu.sync_copy(cnt256, shared_cnt.at[sid])
    plsc.subcore_barrier()

    for l in range(NS):
        pltpu.sync_copy(shared_acc.at[l, pl.ds(sid * SL, SL)], red.at[l])

    def red_body(j, _):
        s = zeros
        for l in range(NS):
            s = s + red[l, pl.ds(j * 16, 16)]
        outbuf[pl.ds(j * 16, 16)] = s
        return 0
    lax.fori_loop(0, SL // 16, red_body, 0)
    pltpu.sync_copy(outbuf, out_sums.at[cid, pl.ds(sid * SL, SL)])

    @pl.when(sid == 0)
    def _():
        for l in range(NS):
            pltpu.sync_copy(shared_cnt.at[l], cntstage.at[l])

        def cb(j, _):
            s = zeros
            for l in range(NS):
                s = s + cntstage[l, pl.ds(j * 16, 16)]
            cnt256[pl.ds(j * 16, 16)] = s
            return 0
        lax.fori_loop(0, HW // 16, cb, 0)
        pltpu.sync_copy(cnt256, out_cnts.at[cid])


def kernel(coords, values):
    sums, cnts = _pool_sc(coords.reshape(-1), values.reshape(-1))
    sums = sums[0] + sums[1]
    cnts = cnts[0] + cnts[1]
    means = sums.reshape(HW, B, C) / cnts[:, None, None]
    return means.transpose(1, 0, 2).reshape(B, HW * C)
